# baseline (speedup 1.0000x reference)
import functools

import jax
import jax.numpy as jnp
from jax import lax
from jax.experimental import pallas as pl
from jax.experimental.pallas import tpu as pltpu
from jax.experimental.pallas import tpu_sc as plsc

N_NODES = 10000
D = 128
N_EDGES = 320000

NC = 2
NS = 16
CPT = D // 2 // NS
CHUNK_W = CPT * N_NODES

EH = N_EDGES // NC
EB = 10000
NB = EH // EB
GROUPS = EB // 16


def _mlp_body(x_ref, w1_ref, b1_ref, w2_ref, b2_ref, w_ref):
    h1 = jnp.dot(x_ref[...], w1_ref[...], preferred_element_type=jnp.float32)
    h1 = jnp.maximum(h1 + b1_ref[...], 0.0)
    ht = lax.dot_general(w2_ref[...], h1, (((0,), (1,)), ((), ())),
                         preferred_element_type=jnp.float32)
    ht = ht + b2_ref[...]
    hb = ht.astype(jnp.bfloat16)
    lo = lax.bitcast_convert_type(hb[:D // 2], jnp.uint16).astype(jnp.uint32)
    hi = lax.bitcast_convert_type(hb[D // 2:], jnp.uint16).astype(jnp.uint32)
    w_ref[...] = lax.bitcast_convert_type(lo | (hi << 16), jnp.float32)


def _mlp_t(node_feature, W1, b1, W2, b2):
    return pl.pallas_call(
        _mlp_body,
        out_shape=jax.ShapeDtypeStruct((D // 2, N_NODES), jnp.float32),
    )(node_feature, W1, b1.reshape(1, D), W2, b2.reshape(D, 1))


_sc_mesh = plsc.VectorSubcoreMesh(core_axis_name="c", subcore_axis_name="s")


@functools.partial(
    pl.kernel,
    out_type=jax.ShapeDtypeStruct((N_EDGES,), jnp.float32),
    mesh=_sc_mesh,
    scratch_types=[
        pltpu.VMEM((CHUNK_W,), jnp.float32),
        pltpu.VMEM((2 * EB,), jnp.int32),
        pltpu.VMEM((2 * EB,), jnp.int32),
        pltpu.VMEM((2 * EB,), jnp.float32),
        pltpu.VMEM((EB,), jnp.int32),
        pltpu.VMEM_SHARED((EH,), jnp.float32),
        pltpu.SemaphoreType.DMA,
        pltpu.SemaphoreType.DMA,
        pltpu.SemaphoreType.DMA,
        pltpu.SemaphoreType.DMA,
    ],
    compiler_params=pltpu.CompilerParams(needs_layout_passes=False),
)
def _sc_gather_dot(h_flat, idx_flat, out_hbm,
                   chunk_v, idx0_v, idx1_v, partial_v, pos_v, acc_sh,
                   sem_c, sem_i0, sem_i1, sem_s):
    half = lax.axis_index("c")
    t = lax.axis_index("s")
    iota16 = lax.iota(jnp.int32, 16)
    ebase0 = half * EH

    pltpu.async_copy(h_flat.at[pl.ds(t * CHUNK_W, CHUNK_W)], chunk_v, sem_c)
    pltpu.async_copy(idx_flat.at[pl.ds(ebase0, EB)], idx0_v.at[pl.ds(0, EB)], sem_i0)
    pltpu.async_copy(idx_flat.at[pl.ds(N_EDGES + ebase0, EB)], idx1_v.at[pl.ds(0, EB)],
                     sem_i1)

    def zbody(j, _):
        partial_v[pl.ds(j * 16, 16)] = jnp.zeros((16,), jnp.float32)
        pos_v[pl.ds(j * 16, 16)] = j * 16 + iota16
        return 0
    lax.fori_loop(0, EB // 16, zbody, 0, unroll=8)
    pltpu.sync_copy(partial_v.at[pl.ds(0, EB)], acc_sh.at[pl.ds(t * EB, EB)])
    plsc.subcore_barrier()
    pltpu.make_async_copy(h_flat.at[pl.ds(t * CHUNK_W, CHUNK_W)], chunk_v,
                          sem_c).wait()

    def block_body(kb, _):
        b = lax.rem(kb, 2)
        ebase = half * EH + kb * EB
        pltpu.make_async_copy(idx_flat.at[pl.ds(ebase, EB)], idx0_v.at[pl.ds(b * EB, EB)],
                              sem_i0).wait()
        pltpu.make_async_copy(idx_flat.at[pl.ds(N_EDGES + ebase, EB)],
                              idx1_v.at[pl.ds(b * EB, EB)], sem_i1).wait()

        @pl.when(kb + 1 < NB)
        def _prefetch():
            nbase = ebase + EB
            pltpu.async_copy(idx_flat.at[pl.ds(nbase, EB)], idx0_v.at[pl.ds((1 - b) * EB, EB)],
                             sem_i0)
            pltpu.async_copy(idx_flat.at[pl.ds(N_EDGES + nbase, EB)],
                             idx1_v.at[pl.ds((1 - b) * EB, EB)], sem_i1)

        boff = b * EB

        @plsc.parallel_loop(0, GROUPS, 1, unroll=8)
        def gbody(g):
            i0 = idx0_v[pl.ds(boff + g * 16, 16)]
            i1 = idx1_v[pl.ds(boff + g * 16, 16)]
            acc = jnp.zeros((16,), jnp.float32)
            for p in range(CPT):
                wa = plsc.load_gather(chunk_v, [i0 + p * N_NODES])
                wb = plsc.load_gather(chunk_v, [i1 + p * N_NODES])
                a0, a1 = plsc.unpack(plsc.bitcast(wa, jnp.bfloat16),
                                     format=plsc.PackFormat.INTERLEAVED)
                b0, b1 = plsc.unpack(plsc.bitcast(wb, jnp.bfloat16),
                                     format=plsc.PackFormat.INTERLEAVED)
                acc = acc + a0 * b0 + a1 * b1
            partial_v[pl.ds(boff + g * 16, 16)] = acc

        @pl.when(kb >= 1)
        def _wait_prev():
            pltpu.make_async_copy(partial_v.at[pl.ds((1 - b) * EB, EB)],
                                  acc_sh.at[pl.ds(0, EB)].at[pos_v],
                                  sem_s).wait()
        pltpu.async_copy(partial_v.at[pl.ds(b * EB, EB)],
                         acc_sh.at[pl.ds(kb * EB, EB)].at[pos_v], sem_s,
                         add=True)
        return 0

    lax.fori_loop(0, NB, block_body, 0)
    pltpu.make_async_copy(partial_v.at[pl.ds(lax.rem(NB - 1, 2) * EB, EB)],
                          acc_sh.at[pl.ds(0, EB)].at[pos_v], sem_s).wait()
    plsc.subcore_barrier()

    pltpu.sync_copy(acc_sh.at[pl.ds(t * EB, EB)], partial_v.at[pl.ds(0, EB)])
    pltpu.sync_copy(partial_v.at[pl.ds(0, EB)],
                    out_hbm.at[pl.ds(half * EH + t * EB, EB)])


def kernel(node_feature, edge_label_index, edge_label, W1, b1, W2, b2):
    h_t = _mlp_t(node_feature, W1, b1, W2, b2)
    pred = _sc_gather_dot(h_t.reshape(-1), edge_label_index.reshape(-1))
    return pred, edge_label


# speedup vs baseline: 1.3573x; 1.3573x over previous
import functools

import jax
import jax.numpy as jnp
from jax import lax
from jax.experimental import pallas as pl
from jax.experimental.pallas import tpu as pltpu
from jax.experimental.pallas import tpu_sc as plsc

N_NODES = 10000
D = 128
N_EDGES = 320000

NC = 2
NS = 16
NW = NC * NS
PW = D // 2

E_PER_T = N_EDGES // NW
B = 400
NBK = E_PER_T // B


def _mlp_body(x_ref, w1_ref, b1_ref, w2_ref, b2_ref, w_ref):
    h1 = jnp.dot(x_ref[...], w1_ref[...], preferred_element_type=jnp.float32)
    h1 = jnp.maximum(h1 + b1_ref[...], 0.0)
    h2 = jnp.dot(h1, w2_ref[...], preferred_element_type=jnp.float32)
    h2 = h2 + b2_ref[...]
    hb = h2.astype(jnp.bfloat16)
    lo = lax.bitcast_convert_type(hb[:, :PW], jnp.uint16).astype(jnp.uint32)
    hi = lax.bitcast_convert_type(hb[:, PW:], jnp.uint16).astype(jnp.uint32)
    w_ref[...] = lax.bitcast_convert_type(lo | (hi << 16), jnp.float32)


def _mlp_packed(node_feature, W1, b1, W2, b2):
    return pl.pallas_call(
        _mlp_body,
        out_shape=jax.ShapeDtypeStruct((N_NODES, PW), jnp.float32),
    )(node_feature, W1, b1.reshape(1, D), W2, b2.reshape(1, D))


_sc_mesh = plsc.VectorSubcoreMesh(core_axis_name="c", subcore_axis_name="s")


@functools.partial(
    pl.kernel,
    out_type=jax.ShapeDtypeStruct((N_EDGES,), jnp.float32),
    mesh=_sc_mesh,
    scratch_types=[
        pltpu.VMEM((2 * B, PW), jnp.float32),
        pltpu.VMEM((2 * B, PW), jnp.float32),
        pltpu.VMEM((2 * B,), jnp.int32),
        pltpu.VMEM((2 * B,), jnp.int32),
        pltpu.VMEM((2 * B,), jnp.float32),
        pltpu.SemaphoreType.DMA,
        pltpu.SemaphoreType.DMA,
        pltpu.SemaphoreType.DMA,
        pltpu.SemaphoreType.DMA,
        pltpu.SemaphoreType.DMA,
    ],
    compiler_params=pltpu.CompilerParams(needs_layout_passes=False,
                                         use_tc_tiling_on_sc=False),
)
def _sc_edge_dot(h_packed, idx_flat, out_hbm,
                 rows0_v, rows1_v, idx0_v, idx1_v, out_v,
                 sem_i0, sem_i1, sem_r0, sem_r1, sem_o):
    t = lax.axis_index("s") * NC + lax.axis_index("c")
    iota16 = lax.iota(jnp.int32, 16)
    last_lane = iota16 == 15
    tbase = t * E_PER_T

    def idx_copies(kb, buf):
        src0 = idx_flat.at[pl.ds(tbase + kb * B, B)]
        src1 = idx_flat.at[pl.ds(N_EDGES + tbase + kb * B, B)]
        d0 = idx0_v.at[pl.ds(buf * B, B)]
        d1 = idx1_v.at[pl.ds(buf * B, B)]
        return ((src0, d0, sem_i0), (src1, d1, sem_i1))

    def row_copies(buf):
        i0 = idx0_v.at[pl.ds(buf * B, B)]
        i1 = idx1_v.at[pl.ds(buf * B, B)]
        return ((h_packed.at[i0], rows0_v.at[pl.ds(buf * B, B)], sem_r0),
                (h_packed.at[i1], rows1_v.at[pl.ds(buf * B, B)], sem_r1))

    for s, d, sem in idx_copies(0, 0):
        pltpu.async_copy(s, d, sem)
    for s, d, sem in idx_copies(0, 0):
        pltpu.make_async_copy(s, d, sem).wait()
    for s, d, sem in row_copies(0):
        pltpu.async_copy(s, d, sem)
    for s, d, sem in idx_copies(1, 1):
        pltpu.async_copy(s, d, sem)

    def block_body(kb, _):
        b = lax.rem(kb, 2)
        nb = 1 - b

        @pl.when(kb + 1 < NBK)
        def _start_next_rows():
            for s, d, sem in idx_copies(kb + 1, nb):
                pltpu.make_async_copy(s, d, sem).wait()
            for s, d, sem in row_copies(nb):
                pltpu.async_copy(s, d, sem)

        for s, d, sem in row_copies(b):
            pltpu.make_async_copy(s, d, sem).wait()

        @pl.when(kb + 2 < NBK)
        def _prefetch_idx():
            for s, d, sem in idx_copies(kb + 2, b):
                pltpu.async_copy(s, d, sem)

        boff = b * B

        @plsc.parallel_loop(0, B, 1, unroll=4)
        def ebody(e):
            acc = jnp.zeros((16,), jnp.float32)
            for c in range(PW // 16):
                wa = rows0_v[boff + e, pl.ds(c * 16, 16)]
                wb = rows1_v[boff + e, pl.ds(c * 16, 16)]
                m = plsc.bitcast(wa, jnp.bfloat16) * plsc.bitcast(
                    wb, jnp.bfloat16)
                m0, m1 = plsc.unpack(m, format=plsc.PackFormat.INTERLEAVED)
                acc = acc + m0 + m1
            tot = plsc.cumsum(acc)
            plsc.store_scatter(out_v, [jnp.full((16,), boff + e, jnp.int32)],
                               tot, mask=last_lane)

        @pl.when(kb >= 1)
        def _wait_prev_out():
            pltpu.make_async_copy(out_v.at[pl.ds(nb * B, B)],
                                  out_hbm.at[pl.ds(tbase, B)], sem_o).wait()
        pltpu.async_copy(out_v.at[pl.ds(boff, B)],
                         out_hbm.at[pl.ds(tbase + kb * B, B)], sem_o)
        return 0

    lax.fori_loop(0, NBK, block_body, 0)
    pltpu.make_async_copy(out_v.at[pl.ds(lax.rem(NBK - 1, 2) * B, B)],
                          out_hbm.at[pl.ds(tbase, B)], sem_o).wait()


def kernel(node_feature, edge_label_index, edge_label, W1, b1, W2, b2):
    h_packed = _mlp_packed(node_feature, W1, b1, W2, b2)
    pred = _sc_edge_dot(h_packed, edge_label_index.reshape(-1))
    return pred, edge_label
